# baseline (speedup 1.0000x reference)
import functools

import jax
import jax.numpy as jnp
from jax import lax
from jax.experimental import pallas as pl
from jax.experimental.pallas import tpu as pltpu
from jax.experimental.pallas import tpu_sc as plsc

VOCAB = 1000000
EMBD = 64
BATCH = 4096
NEGS = 20480
TOT = BATCH + NEGS

NC, NS = 2, 16
NW = NC * NS
ROWS_B = BATCH // NW
ROWS_N = NEGS // NW
IDXW = 128
NCHUNK = ROWS_N // IDXW

CH = 256
GRID = TOT // CH
POS = BATCH // CH
SUB = 4
SROWS = CH // SUB

@functools.cache
def _make_gather3():
    mesh = plsc.VectorSubcoreMesh(core_axis_name="c", subcore_axis_name="s")
    return functools.partial(
        pl.kernel,
        mesh=mesh,
        compiler_params=pltpu.CompilerParams(use_tc_tiling_on_sc=False),
        out_type=(
            jax.ShapeDtypeStruct((TOT, EMBD), jnp.float32),
            jax.ShapeDtypeStruct((BATCH, EMBD), jnp.float32),
        ),
        scratch_types=[
            pltpu.VMEM((ROWS_B,), jnp.int32),
            pltpu.VMEM((ROWS_B,), jnp.int32),
            pltpu.VMEM((ROWS_N,), jnp.int32),
            pltpu.VMEM((2 * ROWS_B + ROWS_N, EMBD), jnp.float32),
            pltpu.SemaphoreType.DMA,
        ],
    )(_gather3_body)


def _gather3_body(emb_hbm, ffw_hbm, inpt_h, trgs_h, rand_h, x_out, e_out,
                  idx_e, idx_c, idx_r, rows, sem):
    wid = lax.axis_index("s") * NC + lax.axis_index("c")
    pltpu.sync_copy(inpt_h.at[pl.ds(wid * ROWS_B, ROWS_B)], idx_e)
    pltpu.sync_copy(trgs_h.at[pl.ds(wid * ROWS_B, ROWS_B)], idx_c)
    pltpu.sync_copy(rand_h.at[pl.ds(wid * ROWS_N, ROWS_N)], idx_r)
    cps = [
        pltpu.async_copy(emb_hbm.at[idx_e], rows.at[pl.ds(0, ROWS_B)], sem),
        pltpu.async_copy(ffw_hbm.at[idx_c], rows.at[pl.ds(ROWS_B, ROWS_B)], sem),
    ]
    for j in range(NCHUNK):
        cps.append(pltpu.async_copy(
            ffw_hbm.at[idx_r.at[pl.ds(j * IDXW, IDXW)]],
            rows.at[pl.ds(2 * ROWS_B + j * IDXW, IDXW)], sem))
    for c in cps:
        c.wait()
    pltpu.sync_copy(rows.at[pl.ds(0, ROWS_B)],
                    e_out.at[pl.ds(wid * ROWS_B, ROWS_B)])
    pltpu.sync_copy(rows.at[pl.ds(ROWS_B, ROWS_B)],
                    x_out.at[pl.ds(wid * ROWS_B, ROWS_B)])
    pltpu.sync_copy(rows.at[pl.ds(2 * ROWS_B, ROWS_N)],
                    x_out.at[pl.ds(BATCH + wid * ROWS_N, ROWS_N)])


CLO = 1.00000005e-07
CHI = 16.11809565095832
W_POS = 1.0 / (BATCH * BATCH)
W_NEG = 1.0 / (NEGS * BATCH)


def _tile_loss(z):
    sp = jnp.maximum(z, 0.0) + jnp.log1p(jnp.exp(-jnp.abs(z)))
    return jnp.sum(jnp.clip(sp, CLO, CHI))


def _chunk_loss(read_tile, chunk_idx):
    is_pos = chunk_idx < POS
    sgn = jnp.where(is_pos, -1.0, 1.0).astype(jnp.float32)
    w = jnp.where(is_pos, W_POS, W_NEG).astype(jnp.float32)
    tot = 0.0
    for k in range(SUB):
        tot += _tile_loss(read_tile(k) * sgn)
    return tot * w


def _loss_body(x_ref, e_ref, o_ref, buf_ref):
    i = pl.program_id(0)
    p = lax.rem(i, 2)

    t = lax.dot_general(x_ref[...], e_ref[...], (((1,), (1,)), ((), ())),
                        preferred_element_type=jnp.float32)

    @pl.when(i == 0)
    def _init():
        o_ref[0, 0] = 0.0

    @pl.when(i > 0)
    def _drain_prev():
        off = (1 - p) * CH
        o_ref[0, 0] += _chunk_loss(
            lambda k: buf_ref[pl.ds(off + k * SROWS, SROWS), :], i - 1)

    buf_ref[pl.ds(p * CH, CH), :] = t

    @pl.when(i == GRID - 1)
    def _drain_last():
        o_ref[0, 0] += _chunk_loss(
            lambda k: t[k * SROWS:(k + 1) * SROWS, :], i)


_loss_call = pl.pallas_call(
    _loss_body,
    grid=(GRID,),
    in_specs=[
        pl.BlockSpec((CH, EMBD), lambda i: (i, 0)),
        pl.BlockSpec((BATCH, EMBD), lambda i: (0, 0)),
    ],
    out_specs=pl.BlockSpec(memory_space=pltpu.SMEM),
    out_shape=jax.ShapeDtypeStruct((1, 1), jnp.float32),
    scratch_shapes=[pltpu.VMEM((2 * CH, BATCH), jnp.float32)],
)


def kernel(inpt, trgs, rand, emb_table, ffw_weight):
    x_all, e_all = _make_gather3()(
        emb_table, ffw_weight,
        inpt.astype(jnp.int32), trgs.astype(jnp.int32), rand.astype(jnp.int32))
    loss = _loss_call(x_all, e_all)
    return loss[0, 0]


# speedup vs baseline: 1.1860x; 1.1860x over previous
import functools

import jax
import jax.numpy as jnp
from jax import lax
from jax.experimental import pallas as pl
from jax.experimental.pallas import tpu as pltpu
from jax.experimental.pallas import tpu_sc as plsc

VOCAB = 1000000
EMBD = 64
BATCH = 4096
NEGS = 20480
TOT = BATCH + NEGS

NC, NS = 2, 16
NW = NC * NS
ROWS_B = BATCH // NW
ROWS_N = NEGS // NW
IDXW = 128
NCHUNK = ROWS_N // IDXW

CH = 256
GRID = TOT // CH
POS = BATCH // CH
SUB = 4
SROWS = CH // SUB

@functools.cache
def _make_gather3():
    mesh = plsc.VectorSubcoreMesh(core_axis_name="c", subcore_axis_name="s")
    return functools.partial(
        pl.kernel,
        mesh=mesh,
        compiler_params=pltpu.CompilerParams(use_tc_tiling_on_sc=False),
        out_type=(
            jax.ShapeDtypeStruct((TOT, EMBD), jnp.float32),
            jax.ShapeDtypeStruct((BATCH, EMBD), jnp.float32),
        ),
        scratch_types=[
            pltpu.VMEM((ROWS_B,), jnp.int32),
            pltpu.VMEM((ROWS_B,), jnp.int32),
            pltpu.VMEM((ROWS_N,), jnp.int32),
            pltpu.VMEM((2 * ROWS_B + ROWS_N, EMBD), jnp.float32),
            pltpu.SemaphoreType.DMA,
        ],
    )(_gather3_body)


def _gather3_body(emb_hbm, ffw_hbm, inpt_h, trgs_h, rand_h, x_out, e_out,
                  idx_e, idx_c, idx_r, rows, sem):
    wid = lax.axis_index("s") * NC + lax.axis_index("c")
    pltpu.sync_copy(inpt_h.at[pl.ds(wid * ROWS_B, ROWS_B)], idx_e)
    pltpu.sync_copy(trgs_h.at[pl.ds(wid * ROWS_B, ROWS_B)], idx_c)
    pltpu.sync_copy(rand_h.at[pl.ds(wid * ROWS_N, ROWS_N)], idx_r)
    cps = [
        pltpu.async_copy(emb_hbm.at[idx_e], rows.at[pl.ds(0, ROWS_B)], sem),
        pltpu.async_copy(ffw_hbm.at[idx_c], rows.at[pl.ds(ROWS_B, ROWS_B)], sem),
    ]
    for j in range(NCHUNK):
        cps.append(pltpu.async_copy(
            ffw_hbm.at[idx_r.at[pl.ds(j * IDXW, IDXW)]],
            rows.at[pl.ds(2 * ROWS_B + j * IDXW, IDXW)], sem))
    for c in cps:
        c.wait()
    pltpu.sync_copy(rows.at[pl.ds(0, ROWS_B)],
                    e_out.at[pl.ds(wid * ROWS_B, ROWS_B)])
    pltpu.sync_copy(rows.at[pl.ds(ROWS_B, ROWS_B)],
                    x_out.at[pl.ds(wid * ROWS_B, ROWS_B)])
    pltpu.sync_copy(rows.at[pl.ds(2 * ROWS_B, ROWS_N)],
                    x_out.at[pl.ds(BATCH + wid * ROWS_N, ROWS_N)])


L2E = 1.4426950408889634
LN2 = 0.6931471805599453
CHI = 16.11809565095832
ZHI2 = CHI * L2E
CLO2 = 1.00000005e-07 / LN2
W_POS = LN2 / (BATCH * BATCH)
W_NEG = LN2 / (NEGS * BATCH)


def _loss_body(x_ref, e_ref, o_ref):
    i = pl.program_id(0)
    is_pos = i < POS
    s = jnp.where(is_pos, -L2E, L2E).astype(jnp.float32)
    w = jnp.where(is_pos, W_POS, W_NEG).astype(jnp.float32)

    t2 = lax.dot_general(x_ref[...] * s, e_ref[...], (((1,), (1,)), ((), ())),
                         preferred_element_type=jnp.float32)

    @pl.when(i == 0)
    def _init():
        o_ref[0, 0] = 0.0

    tot = 0.0
    for k in range(SUB):
        zc = jnp.minimum(t2[k * SROWS:(k + 1) * SROWS, :], ZHI2)
        g = jnp.log2(1.0 + jnp.exp2(zc))
        tot += jnp.sum(jnp.maximum(g, CLO2))
    o_ref[0, 0] += tot * w


_loss_call = pl.pallas_call(
    _loss_body,
    grid=(GRID,),
    in_specs=[
        pl.BlockSpec((CH, EMBD), lambda i: (i, 0)),
        pl.BlockSpec((BATCH, EMBD), lambda i: (0, 0)),
    ],
    out_specs=pl.BlockSpec(memory_space=pltpu.SMEM),
    out_shape=jax.ShapeDtypeStruct((1, 1), jnp.float32),
)


def kernel(inpt, trgs, rand, emb_table, ffw_weight):
    x_all, e_all = _make_gather3()(
        emb_table, ffw_weight,
        inpt.astype(jnp.int32), trgs.astype(jnp.int32), rand.astype(jnp.int32))
    loss = _loss_call(x_all, e_all)
    return loss[0, 0]


# speedup vs baseline: 1.7576x; 1.4820x over previous
import functools

import jax
import jax.numpy as jnp
from jax import lax
from jax.experimental import pallas as pl
from jax.experimental.pallas import tpu as pltpu
from jax.experimental.pallas import tpu_sc as plsc

VOCAB = 1000000
EMBD = 64
LANE = 2 * EMBD
HALFV = VOCAB // 2
BATCH = 4096
NEGS = 20480
TOT = BATCH + NEGS

NC, NS = 2, 16
NW = NC * NS
ROWS_B = BATCH // NW
ROWS_N = NEGS // NW
IDXW = 128
NCHUNK = ROWS_N // IDXW

CH = 256
GRID = TOT // CH
POS = BATCH // CH
SUB = 4
SROWS = CH // SUB

@functools.cache
def _make_gather3():
    mesh = plsc.VectorSubcoreMesh(core_axis_name="c", subcore_axis_name="s")
    return functools.partial(
        pl.kernel,
        mesh=mesh,
        compiler_params=pltpu.CompilerParams(use_tc_tiling_on_sc=True),
        out_type=(
            jax.ShapeDtypeStruct((TOT, LANE), jnp.float32),
            jax.ShapeDtypeStruct((BATCH, LANE), jnp.float32),
        ),
        scratch_types=[
            pltpu.VMEM((ROWS_B,), jnp.int32),
            pltpu.VMEM((ROWS_B,), jnp.int32),
            pltpu.VMEM((ROWS_N,), jnp.int32),
            pltpu.VMEM((2 * ROWS_B + ROWS_N, LANE), jnp.float32),
            pltpu.SemaphoreType.DMA,
        ],
    )(_gather3_body)


def _gather3_body(emb_hbm, ffw_hbm, inpt_h, trgs_h, rand_h, x_out, e_out,
                  idx_e, idx_c, idx_r, rows, sem):
    wid = lax.axis_index("s") * NC + lax.axis_index("c")
    pltpu.sync_copy(inpt_h.at[pl.ds(wid * ROWS_B, ROWS_B)], idx_e)
    pltpu.sync_copy(trgs_h.at[pl.ds(wid * ROWS_B, ROWS_B)], idx_c)
    pltpu.sync_copy(rand_h.at[pl.ds(wid * ROWS_N, ROWS_N)], idx_r)
    cps = [
        pltpu.async_copy(emb_hbm.at[idx_e], rows.at[pl.ds(0, ROWS_B)], sem),
        pltpu.async_copy(ffw_hbm.at[idx_c], rows.at[pl.ds(ROWS_B, ROWS_B)], sem),
    ]
    for j in range(NCHUNK):
        cps.append(pltpu.async_copy(
            ffw_hbm.at[idx_r.at[pl.ds(j * IDXW, IDXW)]],
            rows.at[pl.ds(2 * ROWS_B + j * IDXW, IDXW)], sem))
    for c in cps:
        c.wait()
    pltpu.sync_copy(rows.at[pl.ds(0, ROWS_B)],
                    e_out.at[pl.ds(wid * ROWS_B, ROWS_B)])
    pltpu.sync_copy(rows.at[pl.ds(ROWS_B, ROWS_B)],
                    x_out.at[pl.ds(wid * ROWS_B, ROWS_B)])
    pltpu.sync_copy(rows.at[pl.ds(2 * ROWS_B, ROWS_N)],
                    x_out.at[pl.ds(BATCH + wid * ROWS_N, ROWS_N)])


L2E = 1.4426950408889634
LN2 = 0.6931471805599453
CHI = 16.11809565095832
ZHI2 = CHI * L2E
CLO2 = 1.00000005e-07 / LN2
W_POS = LN2 / (BATCH * BATCH)
W_NEG = LN2 / (NEGS * BATCH)


def _sel_half(lines, par):
    return jnp.where(par > 0.5, lines[:, EMBD:], lines[:, :EMBD])


def _loss_body(x_ref, e_ref, px_ref, pe_ref, o_ref, es_ref):
    i = pl.program_id(0)
    is_pos = i < POS
    s = jnp.where(is_pos, -L2E, L2E).astype(jnp.float32)
    w = jnp.where(is_pos, W_POS, W_NEG).astype(jnp.float32)

    @pl.when(i == 0)
    def _init():
        o_ref[0, 0] = 0.0
        es_ref[...] = _sel_half(e_ref[...], pe_ref[...])

    x = _sel_half(x_ref[...], px_ref[...]) * s
    t2 = lax.dot_general(x, es_ref[...], (((1,), (1,)), ((), ())),
                         preferred_element_type=jnp.float32)

    tot = 0.0
    for k in range(SUB):
        zc = jnp.minimum(t2[k * SROWS:(k + 1) * SROWS, :], ZHI2)
        g = jnp.log2(1.0 + jnp.exp2(zc))
        tot += jnp.sum(jnp.maximum(g, CLO2))
    o_ref[0, 0] += tot * w


_loss_call = pl.pallas_call(
    _loss_body,
    grid=(GRID,),
    in_specs=[
        pl.BlockSpec((CH, LANE), lambda i: (i, 0)),
        pl.BlockSpec((BATCH, LANE), lambda i: (0, 0)),
        pl.BlockSpec((CH, 1), lambda i: (i, 0)),
        pl.BlockSpec((BATCH, 1), lambda i: (0, 0)),
    ],
    out_specs=pl.BlockSpec(memory_space=pltpu.SMEM),
    out_shape=jax.ShapeDtypeStruct((1, 1), jnp.float32),
    scratch_shapes=[pltpu.VMEM((BATCH, EMBD), jnp.float32)],
)


RBN2 = 2048
HSPLIT = ((HALFV + RBN2 - 1) // RBN2) * RBN2
RGRID = HSPLIT // RBN2
_BMAX = (VOCAB - HSPLIT) // RBN2


def _relayout_body(a_ref, b_ref, o_ref):
    o_ref[...] = jnp.concatenate([a_ref[...].T, b_ref[...].T], axis=1)


_relayout_call = pl.pallas_call(
    _relayout_body,
    grid=(RGRID,),
    in_specs=[
        pl.BlockSpec((EMBD, RBN2), lambda i: (0, i)),
        pl.BlockSpec((EMBD, RBN2),
                     lambda i: (0, jnp.minimum(i + RGRID, RGRID + _BMAX))),
    ],
    out_specs=pl.BlockSpec((RBN2, LANE), lambda i: (i, 0)),
    out_shape=jax.ShapeDtypeStruct((HSPLIT, LANE), jnp.float32),
)


def kernel(inpt, trgs, rand, emb_table, ffw_weight):
    embt = emb_table.T
    ffwt = ffw_weight.T
    emb2 = _relayout_call(embt, embt)
    ffw2 = _relayout_call(ffwt, ffwt)
    inpt = inpt.astype(jnp.int32)
    trgs = trgs.astype(jnp.int32)
    rand = rand.astype(jnp.int32)

    def line(v):
        return jnp.where(v < HSPLIT, v, v - HSPLIT)

    x_all, e_all = _make_gather3()(
        emb2, ffw2, line(inpt), line(trgs), line(rand))
    px = jnp.concatenate([trgs, rand]) >= HSPLIT
    pe = inpt >= HSPLIT
    loss = _loss_call(x_all, e_all,
                      px.astype(jnp.float32)[:, None],
                      pe.astype(jnp.float32)[:, None])
    return loss[0, 0]


# speedup vs baseline: 1.7682x; 1.0060x over previous
import functools

import jax
import jax.numpy as jnp
from jax import lax
from jax.experimental import pallas as pl
from jax.experimental.pallas import tpu as pltpu
from jax.experimental.pallas import tpu_sc as plsc

VOCAB = 1000000
EMBD = 64
LANE = 2 * EMBD
HALFV = VOCAB // 2
BATCH = 4096
NEGS = 20480
TOT = BATCH + NEGS

NC, NS = 2, 16
NW = NC * NS
ROWS_B = BATCH // NW
ROWS_N = NEGS // NW
IDXW = 128
NCHUNK = ROWS_N // IDXW

CH = 256
GRID = TOT // CH
POS = BATCH // CH
SUB = 4
SROWS = CH // SUB

@functools.cache
def _make_gather_x():
    mesh = plsc.VectorSubcoreMesh(core_axis_name="c", subcore_axis_name="s")
    return functools.partial(
        pl.kernel,
        mesh=mesh,
        compiler_params=pltpu.CompilerParams(use_tc_tiling_on_sc=True),
        out_type=jax.ShapeDtypeStruct((TOT, LANE), jnp.float32),
        scratch_types=[
            pltpu.VMEM((ROWS_B,), jnp.int32),
            pltpu.VMEM((ROWS_N,), jnp.int32),
            pltpu.VMEM((ROWS_B + ROWS_N, LANE), jnp.float32),
            pltpu.SemaphoreType.DMA,
        ],
    )(_gather_x_body)


def _gather_x_body(ffw_hbm, trgs_h, rand_h, x_out, idx_c, idx_r, rows, sem):
    wid = lax.axis_index("s") * NC + lax.axis_index("c")
    pltpu.sync_copy(trgs_h.at[pl.ds(wid * ROWS_B, ROWS_B)], idx_c)
    pltpu.sync_copy(rand_h.at[pl.ds(wid * ROWS_N, ROWS_N)], idx_r)
    cps = [
        pltpu.async_copy(ffw_hbm.at[idx_c], rows.at[pl.ds(0, ROWS_B)], sem),
    ]
    for j in range(NCHUNK):
        cps.append(pltpu.async_copy(
            ffw_hbm.at[idx_r.at[pl.ds(j * IDXW, IDXW)]],
            rows.at[pl.ds(ROWS_B + j * IDXW, IDXW)], sem))
    for c in cps:
        c.wait()
    pltpu.sync_copy(rows.at[pl.ds(0, ROWS_B)],
                    x_out.at[pl.ds(wid * ROWS_B, ROWS_B)])
    pltpu.sync_copy(rows.at[pl.ds(ROWS_B, ROWS_N)],
                    x_out.at[pl.ds(BATCH + wid * ROWS_N, ROWS_N)])


@functools.cache
def _make_gather_e():
    mesh = plsc.VectorSubcoreMesh(core_axis_name="c", subcore_axis_name="s")
    return functools.partial(
        pl.kernel,
        mesh=mesh,
        compiler_params=pltpu.CompilerParams(use_tc_tiling_on_sc=True),
        out_type=jax.ShapeDtypeStruct((BATCH, LANE), jnp.float32),
        scratch_types=[
            pltpu.VMEM((ROWS_B,), jnp.int32),
            pltpu.VMEM((ROWS_B, LANE), jnp.float32),
            pltpu.SemaphoreType.DMA,
        ],
    )(_gather_e_body)


def _gather_e_body(emb_hbm, inpt_h, e_out, idx_e, rows, sem):
    wid = lax.axis_index("s") * NC + lax.axis_index("c")
    pltpu.sync_copy(inpt_h.at[pl.ds(wid * ROWS_B, ROWS_B)], idx_e)
    pltpu.async_copy(emb_hbm.at[idx_e], rows, sem).wait()
    pltpu.sync_copy(rows, e_out.at[pl.ds(wid * ROWS_B, ROWS_B)])


L2E = 1.4426950408889634
LN2 = 0.6931471805599453
CHI = 16.11809565095832
ZHI2 = CHI * L2E
CLO2 = 1.00000005e-07 / LN2
W_POS = LN2 / (BATCH * BATCH)
W_NEG = LN2 / (NEGS * BATCH)


def _sel_half(lines, par):
    return jnp.where(par > 0.5, lines[:, EMBD:], lines[:, :EMBD])


def _loss_body(x_ref, e_ref, px_ref, pe_ref, o_ref, es_ref):
    i = pl.program_id(0)
    is_pos = i < POS
    s = jnp.where(is_pos, -L2E, L2E).astype(jnp.float32)
    w = jnp.where(is_pos, W_POS, W_NEG).astype(jnp.float32)

    @pl.when(i == 0)
    def _init():
        o_ref[0, 0] = 0.0
        es_ref[...] = _sel_half(e_ref[...], pe_ref[...])

    x = _sel_half(x_ref[...], px_ref[...]) * s
    t2 = lax.dot_general(x, es_ref[...], (((1,), (1,)), ((), ())),
                         preferred_element_type=jnp.float32)

    tot = 0.0
    for k in range(SUB):
        zc = jnp.minimum(t2[k * SROWS:(k + 1) * SROWS, :], ZHI2)
        g = jnp.log2(1.0 + jnp.exp2(zc))
        tot += jnp.sum(jnp.maximum(g, CLO2))
    o_ref[0, 0] += tot * w


_loss_call = pl.pallas_call(
    _loss_body,
    grid=(GRID,),
    in_specs=[
        pl.BlockSpec((CH, LANE), lambda i: (i, 0)),
        pl.BlockSpec((BATCH, LANE), lambda i: (0, 0)),
        pl.BlockSpec((CH, 1), lambda i: (i, 0)),
        pl.BlockSpec((BATCH, 1), lambda i: (0, 0)),
    ],
    out_specs=pl.BlockSpec(memory_space=pltpu.SMEM),
    out_shape=jax.ShapeDtypeStruct((1, 1), jnp.float32),
    scratch_shapes=[pltpu.VMEM((BATCH, EMBD), jnp.float32)],
)


RBN2 = 2048
HSPLIT = ((HALFV + RBN2 - 1) // RBN2) * RBN2
RGRID = HSPLIT // RBN2
_BMAX = (VOCAB - HSPLIT) // RBN2


def _relayout_body(a_ref, b_ref, o_ref):
    o_ref[...] = jnp.concatenate([a_ref[...].T, b_ref[...].T], axis=1)


_relayout_call = pl.pallas_call(
    _relayout_body,
    grid=(RGRID,),
    in_specs=[
        pl.BlockSpec((EMBD, RBN2), lambda i: (0, i)),
        pl.BlockSpec((EMBD, RBN2),
                     lambda i: (0, jnp.minimum(i + RGRID, RGRID + _BMAX))),
    ],
    out_specs=pl.BlockSpec((RBN2, LANE), lambda i: (i, 0)),
    out_shape=jax.ShapeDtypeStruct((HSPLIT, LANE), jnp.float32),
)


def kernel(inpt, trgs, rand, emb_table, ffw_weight):
    inpt = inpt.astype(jnp.int32)
    trgs = trgs.astype(jnp.int32)
    rand = rand.astype(jnp.int32)

    def line(v):
        return jnp.where(v < HSPLIT, v, v - HSPLIT)

    ffwt = ffw_weight.T
    ffw2 = _relayout_call(ffwt, ffwt)
    x_all = _make_gather_x()(ffw2, line(trgs), line(rand))
    embt = emb_table.T
    emb2 = _relayout_call(embt, embt)
    e_all = _make_gather_e()(emb2, line(inpt))
    px = jnp.concatenate([trgs, rand]) >= HSPLIT
    pe = inpt >= HSPLIT
    loss = _loss_call(x_all, e_all,
                      px.astype(jnp.float32)[:, None],
                      pe.astype(jnp.float32)[:, None])
    return loss[0, 0]
